# baseline (speedup 1.0000x reference)
import functools

import jax
import jax.numpy as jnp
from jax import lax
from jax.experimental import pallas as pl
from jax.experimental.pallas import tpu as pltpu
from jax.experimental.pallas import tpu_sc as plsc

_H = (25, 50, 100, 200)
_W = (38, 76, 152, 304)
_SCALES = (1.0 / 32.0, 1.0 / 16.0, 1.0 / 8.0, 1.0 / 4.0)
_OFF = (0, 1900, 9500, 39900)
_HW = tuple(h * w for h, w in zip(_H, _W))
_T0 = 384.0 * 384.0
_T1 = 192.0 * 192.0
_T2 = 96.0 * 96.0

_R_PAD = 1024
_BR = 128
_NC, _NS = 2, 16
_NW = _NC * _NS
_RPW = _R_PAD // _NW


def _make_index_body(n0):
    def body(boxes_ref, idx_ref, w_ref):
        b4 = boxes_ref[...]
        x1 = b4[:, 0:1]
        y1 = b4[:, 1:2]
        x2 = b4[:, 2:3]
        y2 = b4[:, 3:4]
        area = (x2 - x1) * (y2 - y1)
        lvl = jnp.where(
            area >= _T0, 0, jnp.where(area >= _T1, 1, jnp.where(area >= _T2, 2, 3))
        ).astype(jnp.int32)

        def sel_f(vals):
            return jnp.where(
                lvl == 0,
                jnp.float32(vals[0]),
                jnp.where(
                    lvl == 1,
                    jnp.float32(vals[1]),
                    jnp.where(lvl == 2, jnp.float32(vals[2]), jnp.float32(vals[3])),
                ),
            )

        def sel_i(vals):
            return jnp.where(
                lvl == 0,
                jnp.int32(vals[0]),
                jnp.where(
                    lvl == 1,
                    jnp.int32(vals[1]),
                    jnp.where(lvl == 2, jnp.int32(vals[2]), jnp.int32(vals[3])),
                ),
            )

        scale = sel_f(_SCALES)
        hf = sel_f([float(h) for h in _H])
        wf = sel_f([float(w) for w in _W])
        hi_ = sel_i(_H)
        wi_ = sel_i(_W)
        off = sel_i(_OFF)
        hw = sel_i(_HW)

        rid = pl.program_id(0) * _BR + lax.broadcasted_iota(jnp.int32, (_BR, 1), 0)
        base = off + jnp.where(rid >= n0, hw, 0)

        x1s = x1 * scale
        y1s = y1 * scale
        x2s = x2 * scale
        y2s = y2 * scale
        bw = jnp.maximum(x2s - x1s, 1.0) / 7.0
        bh = jnp.maximum(y2s - y1s, 1.0) / 7.0

        col = lax.broadcasted_iota(jnp.int32, (_BR, 784), 1)
        p = col & 1
        d = (col >> 1) & 1
        sxb = (col >> 2) & 1
        syb = (col >> 3) & 1
        binc = col >> 4
        by = (binc * 9363) >> 16
        bxn = binc - by * 7

        syk = (by * 2 + syb).astype(jnp.float32)
        sxk = (bxn * 2 + sxb).astype(jnp.float32)
        ysv = y1s + (syk + 0.5) / 2.0 * bh
        xsv = x1s + (sxk + 0.5) / 2.0 * bw

        vy = (ysv >= -1.0) & (ysv <= hf)
        vx = (xsv >= -1.0) & (xsv <= wf)
        yc = jnp.clip(ysv, 0.0, hf - 1.0)
        xc = jnp.clip(xsv, 0.0, wf - 1.0)
        ly = yc - jnp.floor(yc)
        lx = xc - jnp.floor(xc)
        wy = jnp.where(d == 1, ly, 1.0 - ly)
        wx = jnp.where(p == 1, lx, 1.0 - lx)
        wgt = 0.25 * wy * wx * jnp.where(vy & vx, 1.0, 0.0)
        w_ref[...] = wgt.astype(jnp.float32)

        del hi_
        coli = lax.broadcasted_iota(jnp.int32, (_BR, 196), 1)
        sxi = coli & 1
        syi = (coli >> 1) & 1
        bini = coli >> 2
        byi = (bini * 9363) >> 16
        bxi = bini - byi * 7

        syki = (byi * 2 + syi).astype(jnp.float32)
        sxki = (bxi * 2 + sxi).astype(jnp.float32)
        ysi = y1s + (syki + 0.5) / 2.0 * bh
        xsi = x1s + (sxki + 0.5) / 2.0 * bw
        yci = jnp.clip(ysi, 0.0, hf - 1.0)
        xci = jnp.clip(xsi, 0.0, wf - 1.0)
        y0 = jnp.floor(yci).astype(jnp.int32)
        x0 = jnp.floor(xci).astype(jnp.int32)
        idx_ref[...] = base + y0 * wi_ + x0

    return body


def _index_weights(boxes_p, n0):
    return pl.pallas_call(
        _make_index_body(n0),
        grid=(_R_PAD // _BR,),
        in_specs=[pl.BlockSpec((_BR, 4), lambda i: (i, 0))],
        out_specs=[
            pl.BlockSpec((_BR, 196), lambda i: (i, 0)),
            pl.BlockSpec((_BR, 784), lambda i: (i, 0)),
        ],
        out_shape=[
            jax.ShapeDtypeStruct((_R_PAD, 196), jnp.int32),
            jax.ShapeDtypeStruct((_R_PAD, 784), jnp.float32),
        ],
    )(boxes_p)


def _make_sc_kernel():
    mesh = plsc.VectorSubcoreMesh(core_axis_name="c", subcore_axis_name="s")

    @functools.partial(
        pl.kernel,
        mesh=mesh,
        compiler_params=pltpu.CompilerParams(
            needs_layout_passes=False, use_tc_tiling_on_sc=False
        ),
        out_type=jax.ShapeDtypeStruct((_R_PAD, 49, 128), jnp.float32),
        scratch_types=[
            pltpu.VMEM((2, 7, 28), jnp.int32),
            pltpu.VMEM((2, 98, 128), jnp.float32),
            pltpu.VMEM((2, 28, 256), jnp.float32),
            pltpu.VMEM((2, 49, 128), jnp.float32),
            pltpu.SemaphoreType.DMA,
            pltpu.SemaphoreType.DMA,
            pltpu.SemaphoreType.DMA,
        ],
    )
    def sc(idx_hbm, w_hbm, table_hbm, out_hbm, idx_v, w_v, rows_v, out_v,
           iwsem, gsem0, gsem1):
        wid = lax.axis_index("s") * _NC + lax.axis_index("c")
        base = wid * _RPW
        gsems = (gsem0, gsem1)

        def start_iw(roi, p):
            pltpu.make_async_copy(idx_hbm.at[roi], idx_v.at[p], iwsem).start()
            pltpu.make_async_copy(w_hbm.at[roi], w_v.at[p], iwsem).start()

        def wait_iw(p):
            pltpu.make_async_copy(idx_hbm.at[0], idx_v.at[p], iwsem).wait()
            pltpu.make_async_copy(w_hbm.at[0], w_v.at[p], iwsem).wait()

        def process(roi, p):
            iv = idx_v.at[p]
            wv_ = w_v.at[p]
            ov = out_v.at[p]
            pltpu.make_async_copy(table_hbm.at[iv.at[0]], rows_v.at[0], gsems[0]).start()
            for c in range(7):
                if c + 1 < 7:
                    pltpu.make_async_copy(
                        table_hbm.at[iv.at[c + 1]], rows_v.at[(c + 1) % 2],
                        gsems[(c + 1) % 2]).start()
                pltpu.make_async_copy(
                    table_hbm.at[iv.at[c]], rows_v.at[c % 2], gsems[c % 2]).wait()
                rb = rows_v.at[c % 2]

                def bin_body(i, acc_c, c=c, rb=rb, wv_=wv_, ov=ov):
                    accs = [jnp.zeros((16,), jnp.float32) for _ in range(8)]
                    for s in range(4):
                        lrow = i * 4 + s
                        ws = []
                        for jj in range(4):
                            j = s * 4 + jj
                            ws.append(
                                wv_[14 * c + 2 * i + (j >> 3),
                                    pl.ds((j % 8) * 16, 16)]
                            )
                        for g in range(8):
                            wi0 = plsc.bitcast(rb[lrow, pl.ds(g * 16, 16)], jnp.int32)
                            wi1 = plsc.bitcast(
                                rb[lrow, pl.ds(128 + g * 16, 16)], jnp.int32
                            )
                            p00 = plsc.bitcast(wi0 << 16, jnp.float32)
                            p01 = plsc.bitcast(wi0 & jnp.int32(-65536), jnp.float32)
                            p10 = plsc.bitcast(wi1 << 16, jnp.float32)
                            p11 = plsc.bitcast(wi1 & jnp.int32(-65536), jnp.float32)
                            accs[g] = (accs[g] + ws[0] * p00 + ws[1] * p01
                                       + ws[2] * p10 + ws[3] * p11)
                    for g in range(8):
                        ov[7 * c + i, pl.ds(g * 16, 16)] = accs[g]
                    return acc_c

                lax.fori_loop(0, 7, bin_body, 0)
            pltpu.sync_copy(ov, out_hbm.at[roi])

        start_iw(base, 0)

        def pair(k, carry):
            r0 = base + 2 * k
            wait_iw(0)
            start_iw(r0 + 1, 1)
            process(r0, 0)
            wait_iw(1)
            start_iw(jnp.minimum(r0 + 2, _R_PAD - 1), 0)
            process(r0 + 1, 1)
            return carry

        lax.fori_loop(0, _RPW // 2, pair, 0)
        wait_iw(0)

    return sc


def kernel(feat0, feat1, feat2, feat3, boxes0, boxes1):
    feats = (feat0, feat1, feat2, feat3)
    tabs = []
    for f, wl in zip(feats, _W):
        t = jnp.transpose(f, (0, 2, 3, 1)).reshape(-1, 128).astype(jnp.bfloat16)
        tx = jnp.concatenate([t[1:], jnp.zeros((1, 128), jnp.bfloat16)], axis=0)
        t2 = lax.bitcast_convert_type(jnp.stack([t, tx], axis=-1), jnp.int32)
        ty = jnp.concatenate([t2[wl:], jnp.zeros((wl, 128), jnp.int32)], axis=0)
        tabs.append(jnp.concatenate([t2, ty], axis=1))
    table2 = lax.bitcast_convert_type(jnp.concatenate(tabs, axis=0), jnp.float32)
    n0 = boxes0.shape[0]
    n = n0 + boxes1.shape[0]
    boxes = jnp.concatenate([boxes0, boxes1], axis=0)
    pad = jnp.broadcast_to(
        jnp.array([0.0, 0.0, 16.0, 16.0], jnp.float32), (_R_PAD - n, 4)
    )
    boxes_p = jnp.concatenate([boxes, pad], axis=0)
    idx, w = _index_weights(boxes_p, n0)
    w3 = jnp.broadcast_to(w[:, :, None], (_R_PAD, 784, 16)).reshape(_R_PAD, 98, 128)
    out = _make_sc_kernel()(idx.reshape(_R_PAD, 7, 28), w3, table2)
    out = out[:n].reshape(n, 7, 7, 128)
    return jnp.transpose(out, (0, 3, 1, 2))


# speedup vs baseline: 1.9563x; 1.9563x over previous
import functools

import jax
import jax.numpy as jnp
from jax import lax
from jax.experimental import pallas as pl
from jax.experimental.pallas import tpu as pltpu
from jax.experimental.pallas import tpu_sc as plsc

_H = (25, 50, 100, 200)
_W = (38, 76, 152, 304)
_SCALES = (1.0 / 32.0, 1.0 / 16.0, 1.0 / 8.0, 1.0 / 4.0)
_OFF = (0, 1900, 9500, 39900)
_HW = tuple(h * w for h, w in zip(_H, _W))
_T0 = 384.0 * 384.0
_T1 = 192.0 * 192.0
_T2 = 96.0 * 96.0

_R_PAD = 1024
_BR = 128
_NC, _NS = 2, 16
_NW = _NC * _NS
_RPW = _R_PAD // _NW


def _make_index_body(n0):
    def body(boxes_ref, idx_ref, w_ref):
        b4 = boxes_ref[...]
        x1 = b4[:, 0:1]
        y1 = b4[:, 1:2]
        x2 = b4[:, 2:3]
        y2 = b4[:, 3:4]
        area = (x2 - x1) * (y2 - y1)
        lvl = jnp.where(
            area >= _T0, 0, jnp.where(area >= _T1, 1, jnp.where(area >= _T2, 2, 3))
        ).astype(jnp.int32)

        def sel_f(vals):
            return jnp.where(
                lvl == 0,
                jnp.float32(vals[0]),
                jnp.where(
                    lvl == 1,
                    jnp.float32(vals[1]),
                    jnp.where(lvl == 2, jnp.float32(vals[2]), jnp.float32(vals[3])),
                ),
            )

        def sel_i(vals):
            return jnp.where(
                lvl == 0,
                jnp.int32(vals[0]),
                jnp.where(
                    lvl == 1,
                    jnp.int32(vals[1]),
                    jnp.where(lvl == 2, jnp.int32(vals[2]), jnp.int32(vals[3])),
                ),
            )

        scale = sel_f(_SCALES)
        hf = sel_f([float(h) for h in _H])
        wf = sel_f([float(w) for w in _W])
        hi_ = sel_i(_H)
        wi_ = sel_i(_W)
        off = sel_i(_OFF)
        hw = sel_i(_HW)

        rid = pl.program_id(0) * _BR + lax.broadcasted_iota(jnp.int32, (_BR, 1), 0)
        base = off + jnp.where(rid >= n0, hw, 0)

        x1s = x1 * scale
        y1s = y1 * scale
        x2s = x2 * scale
        y2s = y2 * scale
        bw = jnp.maximum(x2s - x1s, 1.0) / 7.0
        bh = jnp.maximum(y2s - y1s, 1.0) / 7.0

        col = lax.broadcasted_iota(jnp.int32, (_BR, 784), 1)
        cx = col & 1
        cy = (col >> 1) & 1
        sx = (col >> 2) & 1
        sy = (col >> 3) & 1
        binc = col >> 4
        by = (binc * 9363) >> 16
        bxn = binc - by * 7

        syk = (by * 2 + sy).astype(jnp.float32)
        sxk = (bxn * 2 + sx).astype(jnp.float32)
        ysv = y1s + (syk + 0.5) / 2.0 * bh
        xsv = x1s + (sxk + 0.5) / 2.0 * bw

        vy = (ysv >= -1.0) & (ysv <= hf)
        vx = (xsv >= -1.0) & (xsv <= wf)
        yc = jnp.clip(ysv, 0.0, hf - 1.0)
        xc = jnp.clip(xsv, 0.0, wf - 1.0)
        y0f = jnp.floor(yc)
        x0f = jnp.floor(xc)
        ly = yc - y0f
        lx = xc - x0f
        y0 = y0f.astype(jnp.int32)
        x0 = x0f.astype(jnp.int32)
        y1i = jnp.minimum(y0 + 1, hi_ - 1)
        x1i = jnp.minimum(x0 + 1, wi_ - 1)
        ya = jnp.where(cy == 1, y1i, y0)
        xa = jnp.where(cx == 1, x1i, x0)
        wy = jnp.where(cy == 1, ly, 1.0 - ly)
        wx = jnp.where(cx == 1, lx, 1.0 - lx)
        wgt = 0.25 * wy * wx * jnp.where(vy & vx, 1.0, 0.0)

        idx_ref[...] = base + ya * wi_ + xa
        w_ref[...] = wgt.astype(jnp.float32)

    return body


def _index_weights(boxes_p, n0):
    return pl.pallas_call(
        _make_index_body(n0),
        grid=(_R_PAD // _BR,),
        in_specs=[pl.BlockSpec((_BR, 4), lambda i: (i, 0))],
        out_specs=[
            pl.BlockSpec((_BR, 784), lambda i: (i, 0)),
            pl.BlockSpec((_BR, 784), lambda i: (i, 0)),
        ],
        out_shape=[
            jax.ShapeDtypeStruct((_R_PAD, 784), jnp.int32),
            jax.ShapeDtypeStruct((_R_PAD, 784), jnp.float32),
        ],
    )(boxes_p)


def _make_sc_kernel():
    mesh = plsc.VectorSubcoreMesh(core_axis_name="c", subcore_axis_name="s")

    @functools.partial(
        pl.kernel,
        mesh=mesh,
        out_type=jax.ShapeDtypeStruct((_R_PAD, 49, 128), jnp.float32),
        scratch_types=[
            pltpu.VMEM((2, 7, 112), jnp.int32),
            pltpu.VMEM((2, 98, 128), jnp.float32),
            pltpu.VMEM((2, 112, 128), jnp.float32),
            pltpu.VMEM((2, 49, 128), jnp.float32),
            pltpu.SemaphoreType.DMA,
            pltpu.SemaphoreType.DMA,
            pltpu.SemaphoreType.DMA,
        ],
    )
    def sc(idx_hbm, w_hbm, table_hbm, out_hbm, idx_v, w_v, rows_v, out_v,
           iwsem, gsem0, gsem1):
        wid = lax.axis_index("s") * _NC + lax.axis_index("c")
        base = wid * _RPW
        gsems = (gsem0, gsem1)

        def start_iw(roi, p):
            pltpu.make_async_copy(idx_hbm.at[roi], idx_v.at[p], iwsem).start()
            pltpu.make_async_copy(w_hbm.at[roi], w_v.at[p], iwsem).start()

        def wait_iw(p):
            pltpu.make_async_copy(idx_hbm.at[0], idx_v.at[p], iwsem).wait()
            pltpu.make_async_copy(w_hbm.at[0], w_v.at[p], iwsem).wait()

        def process(roi, p):
            iv = idx_v.at[p]
            wv_ = w_v.at[p]
            ov = out_v.at[p]
            pltpu.make_async_copy(table_hbm.at[iv.at[0]], rows_v.at[0], gsems[0]).start()
            for c in range(7):
                if c + 1 < 7:
                    pltpu.make_async_copy(
                        table_hbm.at[iv.at[c + 1]], rows_v.at[(c + 1) % 2],
                        gsems[(c + 1) % 2]).start()
                pltpu.make_async_copy(
                    table_hbm.at[iv.at[c]], rows_v.at[c % 2], gsems[c % 2]).wait()
                rb = rows_v.at[c % 2]

                def bin_body(i, acc_c, c=c, rb=rb, wv_=wv_, ov=ov):
                    accs = [jnp.zeros((16,), jnp.float32) for _ in range(8)]
                    for j in range(16):
                        wvec = wv_[14 * c + 2 * i + (j >> 3), pl.ds((j % 8) * 16, 16)]
                        for v in range(8):
                            accs[v] = accs[v] + wvec * rb[i * 16 + j, pl.ds(v * 16, 16)]
                    for v in range(8):
                        ov[7 * c + i, pl.ds(v * 16, 16)] = accs[v]
                    return acc_c

                lax.fori_loop(0, 7, bin_body, 0)
            pltpu.sync_copy(ov, out_hbm.at[roi])

        start_iw(base, 0)

        def pair(k, carry):
            r0 = base + 2 * k
            wait_iw(0)
            start_iw(r0 + 1, 1)
            process(r0, 0)
            wait_iw(1)
            start_iw(jnp.minimum(r0 + 2, _R_PAD - 1), 0)
            process(r0 + 1, 1)
            return carry

        lax.fori_loop(0, _RPW // 2, pair, 0)
        wait_iw(0)

    return sc


def kernel(feat0, feat1, feat2, feat3, boxes0, boxes1):
    feats = (feat0, feat1, feat2, feat3)
    table = jnp.concatenate(
        [jnp.transpose(f, (0, 2, 3, 1)).reshape(-1, 128) for f in feats], axis=0
    )
    n0 = boxes0.shape[0]
    n = n0 + boxes1.shape[0]
    boxes = jnp.concatenate([boxes0, boxes1], axis=0)
    pad = jnp.broadcast_to(
        jnp.array([0.0, 0.0, 16.0, 16.0], jnp.float32), (_R_PAD - n, 4)
    )
    boxes_p = jnp.concatenate([boxes, pad], axis=0)
    idx, w = _index_weights(boxes_p, n0)
    w3 = jnp.broadcast_to(w[:, :, None], (_R_PAD, 784, 16)).reshape(_R_PAD, 98, 128)
    out = _make_sc_kernel()(idx.reshape(_R_PAD, 7, 112), w3, table)
    out = out[:n].reshape(n, 7, 7, 128)
    return jnp.transpose(out, (0, 3, 1, 2))
